# baseline (speedup 1.0000x reference)
import functools

import jax
import jax.numpy as jnp
from jax import lax
from jax.experimental import pallas as pl
from jax.experimental.pallas import tpu as pltpu
from jax.experimental.pallas import tpu_sc as plsc

NC = 2
NS = 16
L = 16
NW = NC * NS


def _pre_body(x_ref, wa_ref, wb_ref, b1_ref, ua_ref, ub1_ref,
              a_ref, b_ref, xu_ref):
    xb = x_ref[...]
    a_ref[...] = jnp.dot(xb, wa_ref[...], preferred_element_type=jnp.float32)
    b_ref[...] = (jnp.dot(xb, wb_ref[...], preferred_element_type=jnp.float32)
                  + b1_ref[...])
    xu_ref[...] = (jnp.dot(xb, ua_ref[...], preferred_element_type=jnp.float32)
                   + ub1_ref[...])


def _tc_pre(x, w1a, w1b, b1, u1a, ub1, blk):
    n, h = x.shape
    grid = n // blk
    row = pl.BlockSpec((blk, h), lambda i: (i, 0))
    full = pl.BlockSpec((h, h), lambda i: (0, 0))
    vec = pl.BlockSpec((1, h), lambda i: (0, 0))
    out = jax.ShapeDtypeStruct((n, h), jnp.float32)
    return pl.pallas_call(
        _pre_body,
        grid=(grid,),
        in_specs=[row, full, full, vec, full, vec],
        out_specs=[row, row, row],
        out_shape=[out, out, out],
    )(x, w1a, w1b, b1.reshape(1, h), u1a, ub1.reshape(1, h))


def _sc_gather_build(n, e_pad, h, c, nk):
    mesh = plsc.VectorSubcoreMesh(core_axis_name="c", subcore_axis_name="s")

    @functools.partial(
        pl.kernel,
        mesh=mesh,
        out_type=(jax.ShapeDtypeStruct((e_pad, h), jnp.float32),
                  jax.ShapeDtypeStruct((e_pad, h), jnp.float32),
                  jax.ShapeDtypeStruct((e_pad,), jnp.float32)),
        scratch_types=[
            pltpu.VMEM((nk * 2 * c,), jnp.int32),
            [pltpu.VMEM((c, h), jnp.float32) for _ in range(2)],
            [pltpu.VMEM((c, h), jnp.float32) for _ in range(2)],
            [pltpu.VMEM((c,), jnp.float32) for _ in range(2)],
            [[pltpu.VMEM((c,), jnp.float32) for _ in range(6)]
             for _ in range(2)],
            [pltpu.SemaphoreType.DMA for _ in range(4)],
        ],
    )
    def sc_gather(a_hbm, b_hbm, px_hbm, py_hbm, pz_hbm, scc_hbm,
                  pa_hbm, pb_hbm, d2_hbm,
                  idx_v, bufa, bufb, d2_v, coord_v, sems):
        sem_g = sems[0:2]
        sem_w = sems[2:4]
        wid = lax.axis_index("s") * NC + lax.axis_index("c")
        row0 = pl.multiple_of(wid * (nk * 2 * c), 8)
        pltpu.sync_copy(scc_hbm.at[pl.ds(row0, nk * 2 * c)], idx_v)

        def issue(k, s):
            sidx = idx_v.at[pl.ds(k * 2 * c, c)]
            ridx = idx_v.at[pl.ds(k * 2 * c + c, c)]
            d = [pltpu.async_copy(a_hbm.at[sidx], bufa[s], sem_g[s]),
                 pltpu.async_copy(b_hbm.at[ridx], bufb[s], sem_g[s])]
            for t, (tab, idx) in enumerate(
                    ((px_hbm, sidx), (py_hbm, sidx), (pz_hbm, sidx),
                     (px_hbm, ridx), (py_hbm, ridx), (pz_hbm, ridx))):
                d.append(pltpu.async_copy(tab.at[idx], coord_v[s][t],
                                          sem_g[s]))
            return d

        gd = [None, None]
        wd = [None, None]

        def finish(k, s):
            for dd in gd[s]:
                dd.wait()
            for g in range(c // L):
                sl = pl.ds(g * L, L)
                dx = coord_v[s][0][sl] - coord_v[s][3][sl]
                dy = coord_v[s][1][sl] - coord_v[s][4][sl]
                dz = coord_v[s][2][sl] - coord_v[s][5][sl]
                d2_v[s][sl] = dx * dx + dy * dy + dz * dz
            base = (wid * nk + k) * c
            wd[s] = [
                pltpu.async_copy(bufa[s], pa_hbm.at[pl.ds(base, c)],
                                 sem_w[s]),
                pltpu.async_copy(bufb[s], pb_hbm.at[pl.ds(base, c)],
                                 sem_w[s]),
                pltpu.async_copy(d2_v[s], d2_hbm.at[pl.ds(base, c)],
                                 sem_w[s]),
            ]

        for k in range(nk):
            s = k & 1
            if wd[s] is not None:
                for dd in wd[s]:
                    dd.wait()
            gd[s] = issue(k, s)
            if k >= 1:
                finish(k - 1, 1 - s)
        finish(nk - 1, (nk - 1) & 1)
        for s in (0, 1):
            if wd[s] is not None:
                for dd in wd[s]:
                    dd.wait()

    return sc_gather


def _edge_body(pa_ref, pb_ref, d2_ref, w1c_ref, w2_ref, b2_ref, msg_ref):
    pre1 = (pa_ref[...] + pb_ref[...]
            + jnp.sqrt(d2_ref[...]) * w1c_ref[...])
    hmid = pre1 * jax.nn.sigmoid(pre1)
    m = (jnp.dot(hmid, w2_ref[...], preferred_element_type=jnp.float32)
         + b2_ref[...])
    msg_ref[...] = m * jax.nn.sigmoid(m)


def _tc_edge(pa, pb, d2, w1c, w2, b2, blk):
    e, h = pa.shape
    grid = e // blk
    row = pl.BlockSpec((blk, h), lambda i: (i, 0))
    col = pl.BlockSpec((blk, 1), lambda i: (i, 0))
    full = pl.BlockSpec((h, h), lambda i: (0, 0))
    vec = pl.BlockSpec((1, h), lambda i: (0, 0))
    return pl.pallas_call(
        _edge_body,
        grid=(grid,),
        in_specs=[row, row, col, vec, full, vec],
        out_specs=row,
        out_shape=jax.ShapeDtypeStruct((e, h), jnp.float32),
    )(pa, pb, d2.reshape(e, 1), w1c.reshape(1, h), w2, b2.reshape(1, h))


def _sc_scatter_build(n, e, h, c):
    nchunks = e // c
    mrows = -(-nchunks // NS)
    mrows = -(-mrows // 8) * 8 if mrows % 8 else mrows
    cw = h // NC
    rpt = n // NS
    wb = 25
    wbc = 40
    mesh = plsc.VectorSubcoreMesh(core_axis_name="c", subcore_axis_name="s")

    @functools.partial(
        pl.kernel,
        mesh=mesh,
        out_type=jax.ShapeDtypeStruct((n, h), jnp.float32),
        scratch_types=[
            pltpu.VMEM((mrows, c), jnp.int32),
            [pltpu.VMEM((c, cw), jnp.float32) for _ in range(2)],
            pltpu.VMEM((wbc, cw), jnp.float32),
            pltpu.VMEM_SHARED((n, cw), jnp.float32),
            pltpu.SemaphoreType.DMA,
            pltpu.SemaphoreType.DMA,
        ],
    )
    def sc_scatter(msg_hbm, rec2d_hbm, aggr_hbm,
                   idx_v, msg_v, tmp_v, acc_sp, sem0, sem1):
        core = lax.axis_index("c")
        sid = lax.axis_index("s")
        colbase = pl.multiple_of(core * cw, cw)
        start = pl.multiple_of(sid * mrows, 8)
        cnt = jnp.clip(nchunks - sid * mrows, 0, mrows)
        sems = (sem0, sem1)

        pltpu.sync_copy(rec2d_hbm.at[pl.ds(start, mrows)], idx_v)

        def zrow(i, _):
            for j in range(cw // L):
                tmp_v[i, pl.ds(j * L, L)] = jnp.zeros((L,), jnp.float32)
            return 0

        lax.fori_loop(0, wb, zrow, 0)

        def zcopy(t, _):
            pltpu.sync_copy(tmp_v.at[pl.ds(0, wb)],
                            acc_sp.at[pl.ds(sid * rpt + t * wb, wb)])
            return 0

        lax.fori_loop(0, rpt // wb, zcopy, 0)
        plsc.subcore_barrier()

        def msg_slice(k):
            return msg_hbm.at[pl.ds((start + k) * c, c), pl.ds(colbase, cw)]

        @pl.when(cnt > 0)
        def _():
            pltpu.async_copy(msg_slice(0), msg_v[0], sems[0])

        def phase(k, s):
            @pl.when(k < cnt)
            def _():
                @pl.when(k + 1 < cnt)
                def _():
                    pltpu.async_copy(msg_slice(k + 1), msg_v[1 - s],
                                     sems[1 - s])
                pltpu.make_async_copy(msg_slice(k), msg_v[s], sems[s]).wait()
                pltpu.sync_copy(msg_v[s], acc_sp.at[idx_v.at[k]], add=True)

        def pair_body(k2, _):
            phase(2 * k2, 0)
            phase(2 * k2 + 1, 1)
            return 0

        lax.fori_loop(0, (mrows + 1) // 2, pair_body, 0)
        plsc.subcore_barrier()

        nk2 = ((n // wbc) - sid + NS - 1) // NS

        def wb_body(k, _):
            start = (sid + k * NS) * wbc
            pltpu.sync_copy(acc_sp.at[pl.ds(start, wbc)], tmp_v)
            pltpu.sync_copy(tmp_v,
                            aggr_hbm.at[pl.ds(start, wbc), pl.ds(colbase, cw)])
            return 0

        lax.fori_loop(0, nk2, wb_body, 0)

    return sc_scatter


def _node_body(xu_ref, aggr0_ref, aggr1_ref, u1b_ref, u2_ref, ub2_ref,
               out_ref):
    ag = aggr0_ref[...] + aggr1_ref[...]
    u = xu_ref[...] + jnp.dot(ag, u1b_ref[...],
                              preferred_element_type=jnp.float32)
    u = u * jax.nn.sigmoid(u)
    out_ref[...] = (jnp.dot(u, u2_ref[...], preferred_element_type=jnp.float32)
                    + ub2_ref[...])


def _tc_node(xu, aggr0, aggr1, u1b, u2, ub2, blk):
    n, h = xu.shape
    grid = n // blk
    row = pl.BlockSpec((blk, h), lambda i: (i, 0))
    full = pl.BlockSpec((h, h), lambda i: (0, 0))
    vec = pl.BlockSpec((1, h), lambda i: (0, 0))
    return pl.pallas_call(
        _node_body,
        grid=(grid,),
        in_specs=[row, row, row, full, full, vec],
        out_specs=row,
        out_shape=jax.ShapeDtypeStruct((n, h), jnp.float32),
    )(xu, aggr0, aggr1, u1b, u2, ub2.reshape(1, h))


def kernel(x, pos, edge_index, W1, b1, W2, b2, U1, ub1, U2, ub2):
    n, h = x.shape
    e = edge_index.shape[1]
    send = edge_index[0].astype(jnp.int32)
    rec = edge_index[1].astype(jnp.int32)

    w1a = W1[:h]
    w1b = W1[h:2 * h]
    w1c = W1[2 * h]
    u1a = U1[:h]
    u1b = U1[h:]

    a_tab, b_tab, xu = _tc_pre(x, w1a, w1b, b1, u1a, ub1, blk=1000)
    px, py, pz = pos[:, 0], pos[:, 1], pos[:, 2]

    es = e // 2
    cg, nkg = 112, 24
    es_pad = NW * nkg * cg
    pad_idx = jnp.arange(es_pad - es, dtype=jnp.int32) % n
    gather = _sc_gather_build(n, es_pad, h, c=cg, nk=nkg)
    scatter = _sc_scatter_build(n, es, h, c=128)
    aggrs = []
    nch = es // 128
    mrows = -(-nch // NS)
    mrows = -(-mrows // 8) * 8 if mrows % 8 else mrows
    pad_rows = mrows * NS - nch
    for s in range(2):
        sl = slice(s * es, (s + 1) * es)
        send_p = jnp.concatenate([send[sl], pad_idx]).reshape(-1, cg)
        rec_p = jnp.concatenate([rec[sl], pad_idx]).reshape(-1, cg)
        scc = jnp.concatenate([send_p, rec_p], axis=1).reshape(-1)
        pa, pb, d2 = gather(a_tab, b_tab, px, py, pz, scc)
        msg = _tc_edge(pa, pb, d2, w1c, W2, b2, blk=512)
        rec2d = jnp.pad(rec[sl].reshape(nch, 128), ((0, pad_rows), (0, 0)))
        aggrs.append(scatter(msg, rec2d))
    return _tc_node(xu, aggrs[0], aggrs[1], u1b, U2, ub2, blk=1000)


# speedup vs baseline: 1.1335x; 1.1335x over previous
import functools

import jax
import jax.numpy as jnp
from jax import lax
from jax.experimental import pallas as pl
from jax.experimental.pallas import tpu as pltpu
from jax.experimental.pallas import tpu_sc as plsc

NC = 2
NS = 16
L = 16
NW = NC * NS


def _pre_body(x_ref, wa_ref, wb_ref, b1_ref, ua_ref, ub1_ref,
              a_ref, b_ref, xu_ref):
    xb = x_ref[...]
    a_ref[...] = jnp.dot(xb, wa_ref[...], preferred_element_type=jnp.float32)
    b_ref[...] = (jnp.dot(xb, wb_ref[...], preferred_element_type=jnp.float32)
                  + b1_ref[...])
    xu_ref[...] = (jnp.dot(xb, ua_ref[...], preferred_element_type=jnp.float32)
                   + ub1_ref[...])


def _tc_pre(x, w1a, w1b, b1, u1a, ub1, blk):
    n, h = x.shape
    grid = n // blk
    row = pl.BlockSpec((blk, h), lambda i: (i, 0))
    full = pl.BlockSpec((h, h), lambda i: (0, 0))
    vec = pl.BlockSpec((1, h), lambda i: (0, 0))
    out = jax.ShapeDtypeStruct((n, h), jnp.float32)
    return pl.pallas_call(
        _pre_body,
        grid=(grid,),
        in_specs=[row, full, full, vec, full, vec],
        out_specs=[row, row, row],
        out_shape=[out, out, out],
    )(x, w1a, w1b, b1.reshape(1, h), u1a, ub1.reshape(1, h))


def _sc_gather_build(n, e_pad, h, c, nk):
    mesh = plsc.VectorSubcoreMesh(core_axis_name="c", subcore_axis_name="s")

    @functools.partial(
        pl.kernel,
        mesh=mesh,
        out_type=(jax.ShapeDtypeStruct((e_pad, h), jnp.float32),
                  jax.ShapeDtypeStruct((e_pad,), jnp.float32)),
        scratch_types=[
            pltpu.VMEM((nk * 2 * c,), jnp.int32),
            [pltpu.VMEM((c, h), jnp.float32) for _ in range(2)],
            [pltpu.VMEM((c, h), jnp.float32) for _ in range(2)],
            [pltpu.VMEM((c,), jnp.float32) for _ in range(2)],
            [[pltpu.VMEM((c,), jnp.float32) for _ in range(6)]
             for _ in range(2)],
            [pltpu.SemaphoreType.DMA for _ in range(4)],
        ],
    )
    def sc_gather(a_hbm, b_hbm, px_hbm, py_hbm, pz_hbm, scc_hbm,
                  pre0_hbm, d2_hbm,
                  idx_v, bufa, bufb, d2_v, coord_v, sems):
        sem_g = sems[0:2]
        sem_w = sems[2:4]
        wid = lax.axis_index("s") * NC + lax.axis_index("c")
        row0 = pl.multiple_of(wid * (nk * 2 * c), 8)
        pltpu.sync_copy(scc_hbm.at[pl.ds(row0, nk * 2 * c)], idx_v)

        def issue(k, s):
            sidx = idx_v.at[pl.ds(k * 2 * c, c)]
            ridx = idx_v.at[pl.ds(k * 2 * c + c, c)]
            d = [pltpu.async_copy(a_hbm.at[sidx], bufa[s], sem_g[s]),
                 pltpu.async_copy(b_hbm.at[ridx], bufb[s], sem_g[s])]
            for t, (tab, idx) in enumerate(
                    ((px_hbm, sidx), (py_hbm, sidx), (pz_hbm, sidx),
                     (px_hbm, ridx), (py_hbm, ridx), (pz_hbm, ridx))):
                d.append(pltpu.async_copy(tab.at[idx], coord_v[s][t],
                                          sem_g[s]))
            return d

        gd = [None, None]
        wd = [None, None]

        def finish(k, s):
            for dd in gd[s]:
                dd.wait()
            for g in range(c // L):
                sl = pl.ds(g * L, L)
                dx = coord_v[s][0][sl] - coord_v[s][3][sl]
                dy = coord_v[s][1][sl] - coord_v[s][4][sl]
                dz = coord_v[s][2][sl] - coord_v[s][5][sl]
                d2_v[s][sl] = dx * dx + dy * dy + dz * dz

            def row_body(i, _):
                for j in range(h // L):
                    sl = pl.ds(j * L, L)
                    bufa[s][i, sl] = bufa[s][i, sl] + bufb[s][i, sl]
                return 0

            lax.fori_loop(0, c, row_body, 0)
            base = (wid * nk + k) * c
            wd[s] = [
                pltpu.async_copy(bufa[s], pre0_hbm.at[pl.ds(base, c)],
                                 sem_w[s]),
                pltpu.async_copy(d2_v[s], d2_hbm.at[pl.ds(base, c)],
                                 sem_w[s]),
            ]

        for k in range(nk):
            s = k & 1
            if wd[s] is not None:
                for dd in wd[s]:
                    dd.wait()
            gd[s] = issue(k, s)
            if k >= 1:
                finish(k - 1, 1 - s)
        finish(nk - 1, (nk - 1) & 1)
        for s in (0, 1):
            if wd[s] is not None:
                for dd in wd[s]:
                    dd.wait()

    return sc_gather


def _edge_body(pre0_ref, d2_ref, w1c_ref, w2_ref, b2_ref, msg_ref):
    pre1 = pre0_ref[...] + jnp.sqrt(d2_ref[...]) * w1c_ref[...]
    hmid = pre1 * jax.nn.sigmoid(pre1)
    m = (jnp.dot(hmid, w2_ref[...], preferred_element_type=jnp.float32)
         + b2_ref[...])
    msg_ref[...] = m * jax.nn.sigmoid(m)


def _tc_edge(pre0, d2, w1c, w2, b2, blk):
    e, h = pre0.shape
    grid = e // blk
    row = pl.BlockSpec((blk, h), lambda i: (i, 0))
    col = pl.BlockSpec((blk, 1), lambda i: (i, 0))
    full = pl.BlockSpec((h, h), lambda i: (0, 0))
    vec = pl.BlockSpec((1, h), lambda i: (0, 0))
    return pl.pallas_call(
        _edge_body,
        grid=(grid,),
        in_specs=[row, col, vec, full, vec],
        out_specs=row,
        out_shape=jax.ShapeDtypeStruct((e, h), jnp.float32),
    )(pre0, d2.reshape(e, 1), w1c.reshape(1, h), w2, b2.reshape(1, h))


def _sc_scatter_build(n, e, h, c):
    nchunks = e // c
    mrows = -(-nchunks // NS)
    mrows = -(-mrows // 8) * 8 if mrows % 8 else mrows
    cw = h // NC
    rpt = n // NS
    wb = 25
    wbc = 40
    mesh = plsc.VectorSubcoreMesh(core_axis_name="c", subcore_axis_name="s")

    @functools.partial(
        pl.kernel,
        mesh=mesh,
        out_type=jax.ShapeDtypeStruct((n, h), jnp.float32),
        scratch_types=[
            pltpu.VMEM((mrows, c), jnp.int32),
            [pltpu.VMEM((c, cw), jnp.float32) for _ in range(2)],
            pltpu.VMEM((wbc, cw), jnp.float32),
            pltpu.VMEM_SHARED((n, cw), jnp.float32),
            pltpu.SemaphoreType.DMA,
            pltpu.SemaphoreType.DMA,
        ],
    )
    def sc_scatter(msg_hbm, rec2d_hbm, aggr_hbm,
                   idx_v, msg_v, tmp_v, acc_sp, sem0, sem1):
        core = lax.axis_index("c")
        sid = lax.axis_index("s")
        colbase = pl.multiple_of(core * cw, cw)
        start = pl.multiple_of(sid * mrows, 8)
        cnt = jnp.clip(nchunks - sid * mrows, 0, mrows)
        sems = (sem0, sem1)

        pltpu.sync_copy(rec2d_hbm.at[pl.ds(start, mrows)], idx_v)

        def zrow(i, _):
            for j in range(cw // L):
                tmp_v[i, pl.ds(j * L, L)] = jnp.zeros((L,), jnp.float32)
            return 0

        lax.fori_loop(0, wb, zrow, 0)

        def zcopy(t, _):
            pltpu.sync_copy(tmp_v.at[pl.ds(0, wb)],
                            acc_sp.at[pl.ds(sid * rpt + t * wb, wb)])
            return 0

        lax.fori_loop(0, rpt // wb, zcopy, 0)
        plsc.subcore_barrier()

        def msg_slice(k):
            return msg_hbm.at[pl.ds((start + k) * c, c), pl.ds(colbase, cw)]

        @pl.when(cnt > 0)
        def _():
            pltpu.async_copy(msg_slice(0), msg_v[0], sems[0])

        def phase(k, s):
            @pl.when(k < cnt)
            def _():
                @pl.when(k + 1 < cnt)
                def _():
                    pltpu.async_copy(msg_slice(k + 1), msg_v[1 - s],
                                     sems[1 - s])
                pltpu.make_async_copy(msg_slice(k), msg_v[s], sems[s]).wait()
                pltpu.sync_copy(msg_v[s], acc_sp.at[idx_v.at[k]], add=True)

        def pair_body(k2, _):
            phase(2 * k2, 0)
            phase(2 * k2 + 1, 1)
            return 0

        lax.fori_loop(0, (mrows + 1) // 2, pair_body, 0)
        plsc.subcore_barrier()

        nk2 = ((n // wbc) - sid + NS - 1) // NS

        def wb_body(k, _):
            start = (sid + k * NS) * wbc
            pltpu.sync_copy(acc_sp.at[pl.ds(start, wbc)], tmp_v)
            pltpu.sync_copy(tmp_v,
                            aggr_hbm.at[pl.ds(start, wbc), pl.ds(colbase, cw)])
            return 0

        lax.fori_loop(0, nk2, wb_body, 0)

    return sc_scatter


def _node_body(xu_ref, aggr0_ref, aggr1_ref, u1b_ref, u2_ref, ub2_ref,
               out_ref):
    ag = aggr0_ref[...] + aggr1_ref[...]
    u = xu_ref[...] + jnp.dot(ag, u1b_ref[...],
                              preferred_element_type=jnp.float32)
    u = u * jax.nn.sigmoid(u)
    out_ref[...] = (jnp.dot(u, u2_ref[...], preferred_element_type=jnp.float32)
                    + ub2_ref[...])


def _tc_node(xu, aggr0, aggr1, u1b, u2, ub2, blk):
    n, h = xu.shape
    grid = n // blk
    row = pl.BlockSpec((blk, h), lambda i: (i, 0))
    full = pl.BlockSpec((h, h), lambda i: (0, 0))
    vec = pl.BlockSpec((1, h), lambda i: (0, 0))
    return pl.pallas_call(
        _node_body,
        grid=(grid,),
        in_specs=[row, row, row, full, full, vec],
        out_specs=row,
        out_shape=jax.ShapeDtypeStruct((n, h), jnp.float32),
    )(xu, aggr0, aggr1, u1b, u2, ub2.reshape(1, h))


def kernel(x, pos, edge_index, W1, b1, W2, b2, U1, ub1, U2, ub2):
    n, h = x.shape
    e = edge_index.shape[1]
    send = edge_index[0].astype(jnp.int32)
    rec = edge_index[1].astype(jnp.int32)

    w1a = W1[:h]
    w1b = W1[h:2 * h]
    w1c = W1[2 * h]
    u1a = U1[:h]
    u1b = U1[h:]

    a_tab, b_tab, xu = _tc_pre(x, w1a, w1b, b1, u1a, ub1, blk=1000)
    px, py, pz = pos[:, 0], pos[:, 1], pos[:, 2]

    es = e // 2
    cg, nkg = 112, 24
    es_pad = NW * nkg * cg
    pad_idx = jnp.arange(es_pad - es, dtype=jnp.int32) % n
    gather = _sc_gather_build(n, es_pad, h, c=cg, nk=nkg)
    scatter = _sc_scatter_build(n, es, h, c=128)
    aggrs = []
    nch = es // 128
    mrows = -(-nch // NS)
    mrows = -(-mrows // 8) * 8 if mrows % 8 else mrows
    pad_rows = mrows * NS - nch
    for s in range(2):
        sl = slice(s * es, (s + 1) * es)
        send_p = jnp.concatenate([send[sl], pad_idx]).reshape(-1, cg)
        rec_p = jnp.concatenate([rec[sl], pad_idx]).reshape(-1, cg)
        scc = jnp.concatenate([send_p, rec_p], axis=1).reshape(-1)
        pre0, d2 = gather(a_tab, b_tab, px, py, pz, scc)
        msg = _tc_edge(pre0, d2, w1c, W2, b2, blk=512)
        rec2d = jnp.pad(rec[sl].reshape(nch, 128), ((0, pad_rows), (0, 0)))
        aggrs.append(scatter(msg, rec2d))
    return _tc_node(xu, aggrs[0], aggrs[1], u1b, U2, ub2, blk=1000)


# speedup vs baseline: 1.1344x; 1.0009x over previous
import functools

import jax
import jax.numpy as jnp
from jax import lax
from jax.experimental import pallas as pl
from jax.experimental.pallas import tpu as pltpu
from jax.experimental.pallas import tpu_sc as plsc

NC = 2
NS = 16
L = 16
NW = NC * NS


def _pre_body(x_ref, wa_ref, wb_ref, b1_ref, ua_ref, ub1_ref,
              a_ref, b_ref, xu_ref):
    xb = x_ref[...]
    a_ref[...] = jnp.dot(xb, wa_ref[...], preferred_element_type=jnp.float32)
    b_ref[...] = (jnp.dot(xb, wb_ref[...], preferred_element_type=jnp.float32)
                  + b1_ref[...])
    xu_ref[...] = (jnp.dot(xb, ua_ref[...], preferred_element_type=jnp.float32)
                   + ub1_ref[...])


def _tc_pre(x, w1a, w1b, b1, u1a, ub1, blk):
    n, h = x.shape
    grid = n // blk
    row = pl.BlockSpec((blk, h), lambda i: (i, 0))
    full = pl.BlockSpec((h, h), lambda i: (0, 0))
    vec = pl.BlockSpec((1, h), lambda i: (0, 0))
    out = jax.ShapeDtypeStruct((n, h), jnp.float32)
    return pl.pallas_call(
        _pre_body,
        grid=(grid,),
        in_specs=[row, full, full, vec, full, vec],
        out_specs=[row, row, row],
        out_shape=[out, out, out],
    )(x, w1a, w1b, b1.reshape(1, h), u1a, ub1.reshape(1, h))


def _sc_gather_build(n, e_pad, h, c, nk):
    mesh = plsc.VectorSubcoreMesh(core_axis_name="c", subcore_axis_name="s")

    @functools.partial(
        pl.kernel,
        mesh=mesh,
        out_type=(jax.ShapeDtypeStruct((e_pad, h), jnp.float32),
                  jax.ShapeDtypeStruct((e_pad,), jnp.float32)),
        scratch_types=[
            pltpu.VMEM((nk * 2 * c,), jnp.int32),
            [pltpu.VMEM((c, h), jnp.float32) for _ in range(2)],
            [pltpu.VMEM((c, h), jnp.float32) for _ in range(2)],
            [pltpu.VMEM((c,), jnp.float32) for _ in range(2)],
            [[pltpu.VMEM((c,), jnp.float32) for _ in range(6)]
             for _ in range(2)],
            [pltpu.SemaphoreType.DMA for _ in range(4)],
        ],
    )
    def sc_gather(a_hbm, b_hbm, px_hbm, py_hbm, pz_hbm, scc_hbm,
                  pre0_hbm, d2_hbm,
                  idx_v, bufa, bufb, d2_v, coord_v, sems):
        sem_g = sems[0:2]
        sem_w = sems[2:4]
        wid = lax.axis_index("s") * NC + lax.axis_index("c")
        row0 = pl.multiple_of(wid * (nk * 2 * c), 8)
        pltpu.sync_copy(scc_hbm.at[pl.ds(row0, nk * 2 * c)], idx_v)

        def issue(k, s):
            sidx = idx_v.at[pl.ds(k * 2 * c, c)]
            ridx = idx_v.at[pl.ds(k * 2 * c + c, c)]
            d = [pltpu.async_copy(a_hbm.at[sidx], bufa[s], sem_g[s]),
                 pltpu.async_copy(b_hbm.at[ridx], bufb[s], sem_g[s])]
            for t, (tab, idx) in enumerate(
                    ((px_hbm, sidx), (py_hbm, sidx), (pz_hbm, sidx),
                     (px_hbm, ridx), (py_hbm, ridx), (pz_hbm, ridx))):
                d.append(pltpu.async_copy(tab.at[idx], coord_v[s][t],
                                          sem_g[s]))
            return d

        gd = [None, None]
        wd = [None, None]

        def finish(k, s):
            for dd in gd[s]:
                dd.wait()
            for g in range(c // L):
                sl = pl.ds(g * L, L)
                dx = coord_v[s][0][sl] - coord_v[s][3][sl]
                dy = coord_v[s][1][sl] - coord_v[s][4][sl]
                dz = coord_v[s][2][sl] - coord_v[s][5][sl]
                d2_v[s][sl] = dx * dx + dy * dy + dz * dz

            def row_body(i, _):
                for j in range(h // L):
                    sl = pl.ds(j * L, L)
                    bufa[s][i, sl] = bufa[s][i, sl] + bufb[s][i, sl]
                return 0

            lax.fori_loop(0, c, row_body, 0)
            base = (wid * nk + k) * c
            wd[s] = [
                pltpu.async_copy(bufa[s], pre0_hbm.at[pl.ds(base, c)],
                                 sem_w[s]),
                pltpu.async_copy(d2_v[s], d2_hbm.at[pl.ds(base, c)],
                                 sem_w[s]),
            ]

        for k in range(nk):
            s = k & 1
            if wd[s] is not None:
                for dd in wd[s]:
                    dd.wait()
            gd[s] = issue(k, s)
            if k >= 1:
                finish(k - 1, 1 - s)
        finish(nk - 1, (nk - 1) & 1)
        for s in (0, 1):
            if wd[s] is not None:
                for dd in wd[s]:
                    dd.wait()

    return sc_gather


def _edge_body(pre0_ref, d2_ref, w1c_ref, w2_ref, b2_ref, msg_ref):
    pre1 = pre0_ref[...] + jnp.sqrt(d2_ref[...]) * w1c_ref[...]
    hmid = pre1 * jax.nn.sigmoid(pre1)
    m = (jnp.dot(hmid, w2_ref[...], preferred_element_type=jnp.float32)
         + b2_ref[...])
    msg_ref[...] = m * jax.nn.sigmoid(m)


def _tc_edge(pre0, d2, w1c, w2, b2, blk):
    e, h = pre0.shape
    grid = e // blk
    row = pl.BlockSpec((blk, h), lambda i: (i, 0))
    col = pl.BlockSpec((blk, 1), lambda i: (i, 0))
    full = pl.BlockSpec((h, h), lambda i: (0, 0))
    vec = pl.BlockSpec((1, h), lambda i: (0, 0))
    return pl.pallas_call(
        _edge_body,
        grid=(grid,),
        in_specs=[row, col, vec, full, vec],
        out_specs=row,
        out_shape=jax.ShapeDtypeStruct((e, h), jnp.float32),
    )(pre0, d2.reshape(e, 1), w1c.reshape(1, h), w2, b2.reshape(1, h))


def _sc_scatter_build(n, e, h, c):
    nchunks = e // c
    mrows = -(-nchunks // NS)
    mrows = -(-mrows // 8) * 8 if mrows % 8 else mrows
    cw = h // NC
    rpt = n // NS
    wb = 25
    wbc = 40
    mesh = plsc.VectorSubcoreMesh(core_axis_name="c", subcore_axis_name="s")

    @functools.partial(
        pl.kernel,
        mesh=mesh,
        out_type=jax.ShapeDtypeStruct((n, h), jnp.float32),
        scratch_types=[
            pltpu.VMEM((mrows, c), jnp.int32),
            [pltpu.VMEM((c, cw), jnp.float32) for _ in range(2)],
            pltpu.VMEM((wbc, cw), jnp.float32),
            pltpu.VMEM_SHARED((n, cw), jnp.float32),
            pltpu.SemaphoreType.DMA,
            pltpu.SemaphoreType.DMA,
        ],
    )
    def sc_scatter(msg_hbm, rec2d_hbm, aggr_hbm,
                   idx_v, msg_v, tmp_v, acc_sp, sem0, sem1):
        core = lax.axis_index("c")
        sid = lax.axis_index("s")
        colbase = pl.multiple_of(core * cw, cw)
        start = pl.multiple_of(sid * mrows, 8)
        cnt = jnp.clip(nchunks - sid * mrows, 0, mrows)
        sems = (sem0, sem1)

        pltpu.sync_copy(rec2d_hbm.at[pl.ds(start, mrows)], idx_v)

        def zrow(i, _):
            for j in range(cw // L):
                tmp_v[i, pl.ds(j * L, L)] = jnp.zeros((L,), jnp.float32)
            return 0

        lax.fori_loop(0, wb, zrow, 0)

        def zcopy(t, _):
            pltpu.sync_copy(tmp_v.at[pl.ds(0, wb)],
                            acc_sp.at[pl.ds(sid * rpt + t * wb, wb)])
            return 0

        lax.fori_loop(0, rpt // wb, zcopy, 0)
        plsc.subcore_barrier()

        def msg_slice(k):
            return msg_hbm.at[pl.ds((start + k) * c, c), pl.ds(colbase, cw)]

        @pl.when(cnt > 0)
        def _():
            pltpu.async_copy(msg_slice(0), msg_v[0], sems[0])

        def phase(k, s):
            @pl.when(k < cnt)
            def _():
                @pl.when(k + 1 < cnt)
                def _():
                    pltpu.async_copy(msg_slice(k + 1), msg_v[1 - s],
                                     sems[1 - s])
                pltpu.make_async_copy(msg_slice(k), msg_v[s], sems[s]).wait()
                pltpu.sync_copy(msg_v[s], acc_sp.at[idx_v.at[k]], add=True)

        def pair_body(k2, _):
            phase(2 * k2, 0)
            phase(2 * k2 + 1, 1)
            return 0

        lax.fori_loop(0, (mrows + 1) // 2, pair_body, 0)
        plsc.subcore_barrier()

        nk2 = ((n // wbc) - sid + NS - 1) // NS

        def wb_body(k, _):
            start = (sid + k * NS) * wbc
            pltpu.sync_copy(acc_sp.at[pl.ds(start, wbc)], tmp_v)
            pltpu.sync_copy(tmp_v,
                            aggr_hbm.at[pl.ds(start, wbc), pl.ds(colbase, cw)])
            return 0

        lax.fori_loop(0, nk2, wb_body, 0)

    return sc_scatter


def _node_body(xu_ref, aggr0_ref, aggr1_ref, u1b_ref, u2_ref, ub2_ref,
               out_ref):
    ag = aggr0_ref[...] + aggr1_ref[...]
    u = xu_ref[...] + jnp.dot(ag, u1b_ref[...],
                              preferred_element_type=jnp.float32)
    u = u * jax.nn.sigmoid(u)
    out_ref[...] = (jnp.dot(u, u2_ref[...], preferred_element_type=jnp.float32)
                    + ub2_ref[...])


def _tc_node(xu, aggr0, aggr1, u1b, u2, ub2, blk):
    n, h = xu.shape
    grid = n // blk
    row = pl.BlockSpec((blk, h), lambda i: (i, 0))
    full = pl.BlockSpec((h, h), lambda i: (0, 0))
    vec = pl.BlockSpec((1, h), lambda i: (0, 0))
    return pl.pallas_call(
        _node_body,
        grid=(grid,),
        in_specs=[row, row, row, full, full, vec],
        out_specs=row,
        out_shape=jax.ShapeDtypeStruct((n, h), jnp.float32),
    )(xu, aggr0, aggr1, u1b, u2, ub2.reshape(1, h))


def kernel(x, pos, edge_index, W1, b1, W2, b2, U1, ub1, U2, ub2):
    n, h = x.shape
    e = edge_index.shape[1]
    send = edge_index[0].astype(jnp.int32)
    rec = edge_index[1].astype(jnp.int32)

    w1a = W1[:h]
    w1b = W1[h:2 * h]
    w1c = W1[2 * h]
    u1a = U1[:h]
    u1b = U1[h:]

    a_tab, b_tab, xu = _tc_pre(x, w1a, w1b, b1, u1a, ub1, blk=1000)
    px, py, pz = pos[:, 0], pos[:, 1], pos[:, 2]

    es = e // 2
    cg, nkg = 112, 24
    es_pad = NW * nkg * cg
    pad_idx = jnp.arange(es_pad - es, dtype=jnp.int32) % n
    gather = _sc_gather_build(n, es_pad, h, c=cg, nk=nkg)
    scatter = _sc_scatter_build(n, es, h, c=128)
    aggrs = []
    nch = es // 128
    mrows = -(-nch // NS)
    mrows = -(-mrows // 8) * 8 if mrows % 8 else mrows
    pad_rows = mrows * NS - nch
    sccs, rec2ds = [], []
    for s in range(2):
        sl = slice(s * es, (s + 1) * es)
        send_p = jnp.concatenate([send[sl], pad_idx]).reshape(-1, cg)
        rec_p = jnp.concatenate([rec[sl], pad_idx]).reshape(-1, cg)
        sccs.append(jnp.concatenate([send_p, rec_p], axis=1).reshape(-1))
        rec2ds.append(
            jnp.pad(rec[sl].reshape(nch, 128), ((0, pad_rows), (0, 0))))
    sccs = [jax.lax.optimization_barrier(s) for s in sccs]
    rec2ds = [jax.lax.optimization_barrier(r) for r in rec2ds]
    for s in range(2):
        pre0, d2 = gather(a_tab, b_tab, px, py, pz, sccs[s])
        msg = _tc_edge(pre0, d2, w1c, W2, b2, blk=512)
        aggrs.append(scatter(msg, rec2ds[s]))
    return _tc_node(xu, aggrs[0], aggrs[1], u1b, U2, ub2, blk=1000)


# speedup vs baseline: 1.1474x; 1.0114x over previous
import functools

import jax
import jax.numpy as jnp
from jax import lax
from jax.experimental import pallas as pl
from jax.experimental.pallas import tpu as pltpu
from jax.experimental.pallas import tpu_sc as plsc

NC = 2
NS = 16
L = 16
NW = NC * NS


def _pre_body(x_ref, wa_ref, wb_ref, b1_ref, ua_ref, ub1_ref,
              a_ref, b_ref, xu_ref):
    xb = x_ref[...]
    a_ref[...] = jnp.dot(xb, wa_ref[...], preferred_element_type=jnp.float32)
    b_ref[...] = (jnp.dot(xb, wb_ref[...], preferred_element_type=jnp.float32)
                  + b1_ref[...])
    xu_ref[...] = (jnp.dot(xb, ua_ref[...], preferred_element_type=jnp.float32)
                   + ub1_ref[...])


def _tc_pre(x, w1a, w1b, b1, u1a, ub1, blk):
    n, h = x.shape
    grid = n // blk
    row = pl.BlockSpec((blk, h), lambda i: (i, 0))
    full = pl.BlockSpec((h, h), lambda i: (0, 0))
    vec = pl.BlockSpec((1, h), lambda i: (0, 0))
    out = jax.ShapeDtypeStruct((n, h), jnp.float32)
    return pl.pallas_call(
        _pre_body,
        grid=(grid,),
        in_specs=[row, full, full, vec, full, vec],
        out_specs=[row, row, row],
        out_shape=[out, out, out],
    )(x, w1a, w1b, b1.reshape(1, h), u1a, ub1.reshape(1, h))


def _sc_gather_build(n, e_pad, h, c, nk):
    mesh = plsc.VectorSubcoreMesh(core_axis_name="c", subcore_axis_name="s")

    @functools.partial(
        pl.kernel,
        mesh=mesh,
        out_type=(jax.ShapeDtypeStruct((e_pad, h), jnp.float32),
                  jax.ShapeDtypeStruct((e_pad,), jnp.float32)),
        scratch_types=[
            pltpu.VMEM((nk * 2 * c,), jnp.int32),
            [pltpu.VMEM((c, h), jnp.float32) for _ in range(2)],
            [pltpu.VMEM((c, h), jnp.float32) for _ in range(2)],
            [pltpu.VMEM((c,), jnp.float32) for _ in range(2)],
            [[pltpu.VMEM((c,), jnp.float32) for _ in range(6)]
             for _ in range(2)],
            [pltpu.SemaphoreType.DMA for _ in range(4)],
        ],
    )
    def sc_gather(a_hbm, b_hbm, px_hbm, py_hbm, pz_hbm, scc_hbm,
                  pre0_hbm, d2_hbm,
                  idx_v, bufa, bufb, d2_v, coord_v, sems):
        sem_g = sems[0:2]
        sem_w = sems[2:4]
        wid = lax.axis_index("s") * NC + lax.axis_index("c")
        row0 = pl.multiple_of(wid * (nk * 2 * c), 8)
        pltpu.sync_copy(scc_hbm.at[pl.ds(row0, nk * 2 * c)], idx_v)

        def issue(k, s):
            sidx = idx_v.at[pl.ds(k * 2 * c, c)]
            ridx = idx_v.at[pl.ds(k * 2 * c + c, c)]
            d = [pltpu.async_copy(a_hbm.at[sidx], bufa[s], sem_g[s]),
                 pltpu.async_copy(b_hbm.at[ridx], bufb[s], sem_g[s])]
            for t, (tab, idx) in enumerate(
                    ((px_hbm, sidx), (py_hbm, sidx), (pz_hbm, sidx),
                     (px_hbm, ridx), (py_hbm, ridx), (pz_hbm, ridx))):
                d.append(pltpu.async_copy(tab.at[idx], coord_v[s][t],
                                          sem_g[s]))
            return d

        gd = [None, None]
        wd = [None, None]

        def finish(k, s):
            for dd in gd[s]:
                dd.wait()
            for g in range(c // L):
                sl = pl.ds(g * L, L)
                dx = coord_v[s][0][sl] - coord_v[s][3][sl]
                dy = coord_v[s][1][sl] - coord_v[s][4][sl]
                dz = coord_v[s][2][sl] - coord_v[s][5][sl]
                d2_v[s][sl] = dx * dx + dy * dy + dz * dz

            def row_body(i, _):
                for j in range(h // L):
                    sl = pl.ds(j * L, L)
                    bufa[s][i, sl] = bufa[s][i, sl] + bufb[s][i, sl]
                return 0

            lax.fori_loop(0, c, row_body, 0)
            base = (wid * nk + k) * c
            wd[s] = [
                pltpu.async_copy(bufa[s], pre0_hbm.at[pl.ds(base, c)],
                                 sem_w[s]),
                pltpu.async_copy(d2_v[s], d2_hbm.at[pl.ds(base, c)],
                                 sem_w[s]),
            ]

        for k in range(nk):
            s = k & 1
            if wd[s] is not None:
                for dd in wd[s]:
                    dd.wait()
            gd[s] = issue(k, s)
            if k >= 1:
                finish(k - 1, 1 - s)
        finish(nk - 1, (nk - 1) & 1)
        for s in (0, 1):
            if wd[s] is not None:
                for dd in wd[s]:
                    dd.wait()

    return sc_gather


def _edge_body(pre0_ref, d2_ref, w1c_ref, w2_ref, b2_ref, msg_ref):
    pre1 = pre0_ref[...] + jnp.sqrt(d2_ref[...]) * w1c_ref[...]
    hmid = pre1 * jax.nn.sigmoid(pre1)
    m = (jnp.dot(hmid.astype(jnp.bfloat16), w2_ref[...],
                 preferred_element_type=jnp.float32) + b2_ref[...])
    msg_ref[...] = m * jax.nn.sigmoid(m)


def _tc_edge(pre0, d2, w1c, w2, b2, blk):
    e, h = pre0.shape
    grid = e // blk
    row = pl.BlockSpec((blk, h), lambda i: (i, 0))
    col = pl.BlockSpec((blk, 1), lambda i: (i, 0))
    full = pl.BlockSpec((h, h), lambda i: (0, 0))
    vec = pl.BlockSpec((1, h), lambda i: (0, 0))
    return pl.pallas_call(
        _edge_body,
        grid=(grid,),
        in_specs=[row, col, vec, full, vec],
        out_specs=row,
        out_shape=jax.ShapeDtypeStruct((e, h), jnp.float32),
    )(pre0, d2.reshape(e, 1), w1c.reshape(1, h),
      w2.astype(jnp.bfloat16), b2.reshape(1, h))


def _sc_scatter_build(n, e, h, c):
    nchunks = e // c
    mrows = -(-nchunks // NS)
    mrows = -(-mrows // 8) * 8 if mrows % 8 else mrows
    cw = h // NC
    rpt = n // NS
    wb = 25
    wbc = 40
    mesh = plsc.VectorSubcoreMesh(core_axis_name="c", subcore_axis_name="s")

    @functools.partial(
        pl.kernel,
        mesh=mesh,
        out_type=jax.ShapeDtypeStruct((n, h), jnp.float32),
        scratch_types=[
            pltpu.VMEM((mrows, c), jnp.int32),
            [pltpu.VMEM((c, cw), jnp.float32) for _ in range(2)],
            pltpu.VMEM((wbc, cw), jnp.float32),
            pltpu.VMEM_SHARED((n, cw), jnp.float32),
            pltpu.SemaphoreType.DMA,
            pltpu.SemaphoreType.DMA,
        ],
    )
    def sc_scatter(msg_hbm, rec2d_hbm, aggr_hbm,
                   idx_v, msg_v, tmp_v, acc_sp, sem0, sem1):
        core = lax.axis_index("c")
        sid = lax.axis_index("s")
        colbase = pl.multiple_of(core * cw, cw)
        start = pl.multiple_of(sid * mrows, 8)
        cnt = jnp.clip(nchunks - sid * mrows, 0, mrows)
        sems = (sem0, sem1)

        pltpu.sync_copy(rec2d_hbm.at[pl.ds(start, mrows)], idx_v)

        def zrow(i, _):
            for j in range(cw // L):
                tmp_v[i, pl.ds(j * L, L)] = jnp.zeros((L,), jnp.float32)
            return 0

        lax.fori_loop(0, wb, zrow, 0)

        def zcopy(t, _):
            pltpu.sync_copy(tmp_v.at[pl.ds(0, wb)],
                            acc_sp.at[pl.ds(sid * rpt + t * wb, wb)])
            return 0

        lax.fori_loop(0, rpt // wb, zcopy, 0)
        plsc.subcore_barrier()

        def msg_slice(k):
            return msg_hbm.at[pl.ds((start + k) * c, c), pl.ds(colbase, cw)]

        @pl.when(cnt > 0)
        def _():
            pltpu.async_copy(msg_slice(0), msg_v[0], sems[0])

        def phase(k, s):
            @pl.when(k < cnt)
            def _():
                @pl.when(k + 1 < cnt)
                def _():
                    pltpu.async_copy(msg_slice(k + 1), msg_v[1 - s],
                                     sems[1 - s])
                pltpu.make_async_copy(msg_slice(k), msg_v[s], sems[s]).wait()
                pltpu.sync_copy(msg_v[s], acc_sp.at[idx_v.at[k]], add=True)

        def pair_body(k2, _):
            phase(2 * k2, 0)
            phase(2 * k2 + 1, 1)
            return 0

        lax.fori_loop(0, (mrows + 1) // 2, pair_body, 0)
        plsc.subcore_barrier()

        nk2 = ((n // wbc) - sid + NS - 1) // NS

        def wb_body(k, _):
            start = (sid + k * NS) * wbc
            pltpu.sync_copy(acc_sp.at[pl.ds(start, wbc)], tmp_v)
            pltpu.sync_copy(tmp_v,
                            aggr_hbm.at[pl.ds(start, wbc), pl.ds(colbase, cw)])
            return 0

        lax.fori_loop(0, nk2, wb_body, 0)

    return sc_scatter


def _node_body(xu_ref, aggr0_ref, aggr1_ref, u1b_ref, u2_ref, ub2_ref,
               out_ref):
    ag = aggr0_ref[...] + aggr1_ref[...]
    u = xu_ref[...] + jnp.dot(ag, u1b_ref[...],
                              preferred_element_type=jnp.float32)
    u = u * jax.nn.sigmoid(u)
    out_ref[...] = (jnp.dot(u, u2_ref[...], preferred_element_type=jnp.float32)
                    + ub2_ref[...])


def _tc_node(xu, aggr0, aggr1, u1b, u2, ub2, blk):
    n, h = xu.shape
    grid = n // blk
    row = pl.BlockSpec((blk, h), lambda i: (i, 0))
    full = pl.BlockSpec((h, h), lambda i: (0, 0))
    vec = pl.BlockSpec((1, h), lambda i: (0, 0))
    return pl.pallas_call(
        _node_body,
        grid=(grid,),
        in_specs=[row, row, row, full, full, vec],
        out_specs=row,
        out_shape=jax.ShapeDtypeStruct((n, h), jnp.float32),
    )(xu, aggr0, aggr1, u1b, u2, ub2.reshape(1, h))


def kernel(x, pos, edge_index, W1, b1, W2, b2, U1, ub1, U2, ub2):
    n, h = x.shape
    e = edge_index.shape[1]
    send = edge_index[0].astype(jnp.int32)
    rec = edge_index[1].astype(jnp.int32)

    w1a = W1[:h]
    w1b = W1[h:2 * h]
    w1c = W1[2 * h]
    u1a = U1[:h]
    u1b = U1[h:]

    a_tab, b_tab, xu = _tc_pre(x, w1a, w1b, b1, u1a, ub1, blk=1000)
    px, py, pz = pos[:, 0], pos[:, 1], pos[:, 2]

    es = e // 2
    cg, nkg = 112, 24
    es_pad = NW * nkg * cg
    pad_idx = jnp.arange(es_pad - es, dtype=jnp.int32) % n
    gather = _sc_gather_build(n, es_pad, h, c=cg, nk=nkg)
    scatter = _sc_scatter_build(n, es, h, c=128)
    aggrs = []
    nch = es // 128
    mrows = -(-nch // NS)
    mrows = -(-mrows // 8) * 8 if mrows % 8 else mrows
    pad_rows = mrows * NS - nch
    sccs, rec2ds = [], []
    for s in range(2):
        sl = slice(s * es, (s + 1) * es)
        send_p = jnp.concatenate([send[sl], pad_idx]).reshape(-1, cg)
        rec_p = jnp.concatenate([rec[sl], pad_idx]).reshape(-1, cg)
        sccs.append(jnp.concatenate([send_p, rec_p], axis=1).reshape(-1))
        rec2ds.append(
            jnp.pad(rec[sl].reshape(nch, 128), ((0, pad_rows), (0, 0))))
    sccs = [jax.lax.optimization_barrier(s) for s in sccs]
    rec2ds = [jax.lax.optimization_barrier(r) for r in rec2ds]
    for s in range(2):
        pre0, d2 = gather(a_tab, b_tab, px, py, pz, sccs[s])
        msg = _tc_edge(pre0, d2, w1c, W2, b2, blk=512)
        aggrs.append(scatter(msg, rec2ds[s]))
    return _tc_node(xu, aggrs[0], aggrs[1], u1b, U2, ub2, blk=1000)


# speedup vs baseline: 1.2231x; 1.0660x over previous
import functools

import jax
import jax.numpy as jnp
from jax import lax
from jax.experimental import pallas as pl
from jax.experimental.pallas import tpu as pltpu
from jax.experimental.pallas import tpu_sc as plsc

NC = 2
NS = 16
L = 16
NW = NC * NS


def _pre_body(x_ref, wa_ref, wb_ref, b1_ref, ua_ref, ub1_ref,
              a_ref, b_ref, xu_ref):
    xb = x_ref[...]
    a_ref[...] = jnp.dot(xb, wa_ref[...], preferred_element_type=jnp.float32)
    b_ref[...] = (jnp.dot(xb, wb_ref[...], preferred_element_type=jnp.float32)
                  + b1_ref[...])
    xu_ref[...] = (jnp.dot(xb, ua_ref[...], preferred_element_type=jnp.float32)
                   + ub1_ref[...])


def _tc_pre(x, w1a, w1b, b1, u1a, ub1, blk):
    n, h = x.shape
    grid = n // blk
    row = pl.BlockSpec((blk, h), lambda i: (i, 0))
    full = pl.BlockSpec((h, h), lambda i: (0, 0))
    vec = pl.BlockSpec((1, h), lambda i: (0, 0))
    out = jax.ShapeDtypeStruct((n, h), jnp.float32)
    return pl.pallas_call(
        _pre_body,
        grid=(grid,),
        in_specs=[row, full, full, vec, full, vec],
        out_specs=[row, row, row],
        out_shape=[out, out, out],
    )(x, w1a, w1b, b1.reshape(1, h), u1a, ub1.reshape(1, h))


def _sc_gather_build(n, e, h, c):
    nchunks = e // c
    mesh = plsc.VectorSubcoreMesh(core_axis_name="c", subcore_axis_name="s")

    @functools.partial(
        pl.kernel,
        mesh=mesh,
        out_type=(jax.ShapeDtypeStruct((e, h), jnp.float32),
                  jax.ShapeDtypeStruct((e,), jnp.float32)),
        scratch_types=[
            pltpu.VMEM((c,), jnp.int32),
            pltpu.VMEM((c,), jnp.int32),
            pltpu.VMEM((c, h), jnp.float32),
            pltpu.VMEM((c, h), jnp.float32),
            pltpu.VMEM((c,), jnp.float32),
            [pltpu.VMEM((c,), jnp.float32) for _ in range(6)],
            pltpu.SemaphoreType.DMA,
            pltpu.SemaphoreType.DMA,
            pltpu.SemaphoreType.DMA,
        ],
    )
    def sc_gather(a_hbm, b_hbm, px_hbm, py_hbm, pz_hbm, send_hbm, rec_hbm,
                  pre0_hbm, d2_hbm,
                  sidx_v, ridx_v, bufa, bufb, d2_v, coord_v,
                  sem_a, sem_b, sem_p):
        wid = lax.axis_index("s") * NC + lax.axis_index("c")
        nk = (nchunks - wid + NW - 1) // NW

        def chunk_body(k, _):
            cid = wid + k * NW
            base = cid * c
            pltpu.sync_copy(send_hbm.at[pl.ds(base, c)], sidx_v)
            pltpu.sync_copy(rec_hbm.at[pl.ds(base, c)], ridx_v)
            cp_a = pltpu.async_copy(a_hbm.at[sidx_v], bufa, sem_a)
            cp_b = pltpu.async_copy(b_hbm.at[ridx_v], bufb, sem_b)
            cps = []
            for t, (tab, idx) in enumerate(
                    ((px_hbm, sidx_v), (py_hbm, sidx_v), (pz_hbm, sidx_v),
                     (px_hbm, ridx_v), (py_hbm, ridx_v), (pz_hbm, ridx_v))):
                cps.append(pltpu.async_copy(tab.at[idx], coord_v[t], sem_p))
            cp_a.wait()
            cp_b.wait()

            def row_body(i, _):
                for j in range(h // L):
                    sl = pl.ds(j * L, L)
                    bufa[i, sl] = bufa[i, sl] + bufb[i, sl]
                return 0

            lax.fori_loop(0, c, row_body, 0)

            for cp in cps:
                cp.wait()
            for g in range(c // L):
                sl = pl.ds(g * L, L)
                dx = coord_v[0][sl] - coord_v[3][sl]
                dy = coord_v[1][sl] - coord_v[4][sl]
                dz = coord_v[2][sl] - coord_v[5][sl]
                d2_v[sl] = dx * dx + dy * dy + dz * dz

            pltpu.sync_copy(bufa, pre0_hbm.at[pl.ds(base, c)])
            pltpu.sync_copy(d2_v, d2_hbm.at[pl.ds(base, c)])
            return 0

        lax.fori_loop(0, nk, chunk_body, 0)

    return sc_gather


def _edge_body(pre0_ref, d2_ref, w1c_ref, w2_ref, b2_ref, msg_ref):
    pre1 = (pre0_ref[...].astype(jnp.float32)
            + jnp.sqrt(d2_ref[...]) * w1c_ref[...])
    hmid = pre1 * jax.nn.sigmoid(pre1)
    m = (jnp.dot(hmid.astype(jnp.bfloat16), w2_ref[...],
                 preferred_element_type=jnp.float32) + b2_ref[...])
    msg_ref[...] = m * jax.nn.sigmoid(m)


def _tc_edge(pre0, d2, w1c, w2, b2, blk):
    e, h = pre0.shape
    grid = e // blk
    row = pl.BlockSpec((blk, h), lambda i: (i, 0))
    col = pl.BlockSpec((blk, 1), lambda i: (i, 0))
    full = pl.BlockSpec((h, h), lambda i: (0, 0))
    vec = pl.BlockSpec((1, h), lambda i: (0, 0))
    return pl.pallas_call(
        _edge_body,
        grid=(grid,),
        in_specs=[row, col, vec, full, vec],
        out_specs=row,
        out_shape=jax.ShapeDtypeStruct((e, h), jnp.float32),
    )(pre0, d2.reshape(e, 1), w1c.reshape(1, h),
      w2.astype(jnp.bfloat16), b2.reshape(1, h))


def _sc_scatter_build(n, e, h, c):
    nchunks = e // c
    mrows = -(-nchunks // NS)
    mrows = -(-mrows // 8) * 8 if mrows % 8 else mrows
    cw = h // NC
    rpt = n // NS
    wb = 25
    wbc = 40
    mesh = plsc.VectorSubcoreMesh(core_axis_name="c", subcore_axis_name="s")

    @functools.partial(
        pl.kernel,
        mesh=mesh,
        out_type=jax.ShapeDtypeStruct((n, h), jnp.float32),
        scratch_types=[
            pltpu.VMEM((mrows, c), jnp.int32),
            [pltpu.VMEM((c, cw), jnp.float32) for _ in range(2)],
            pltpu.VMEM((wbc, cw), jnp.float32),
            pltpu.VMEM_SHARED((n, cw), jnp.float32),
            pltpu.SemaphoreType.DMA,
            pltpu.SemaphoreType.DMA,
        ],
    )
    def sc_scatter(msg_hbm, rec2d_hbm, aggr_hbm,
                   idx_v, msg_v, tmp_v, acc_sp, sem0, sem1):
        core = lax.axis_index("c")
        sid = lax.axis_index("s")
        colbase = pl.multiple_of(core * cw, cw)
        start = pl.multiple_of(sid * mrows, 8)
        cnt = jnp.clip(nchunks - sid * mrows, 0, mrows)
        sems = (sem0, sem1)

        pltpu.sync_copy(rec2d_hbm.at[pl.ds(start, mrows)], idx_v)

        def zrow(i, _):
            for j in range(cw // L):
                tmp_v[i, pl.ds(j * L, L)] = jnp.zeros((L,), jnp.float32)
            return 0

        lax.fori_loop(0, wb, zrow, 0)

        def zcopy(t, _):
            pltpu.sync_copy(tmp_v.at[pl.ds(0, wb)],
                            acc_sp.at[pl.ds(sid * rpt + t * wb, wb)])
            return 0

        lax.fori_loop(0, rpt // wb, zcopy, 0)
        plsc.subcore_barrier()

        def msg_slice(k):
            return msg_hbm.at[pl.ds((start + k) * c, c), pl.ds(colbase, cw)]

        @pl.when(cnt > 0)
        def _():
            pltpu.async_copy(msg_slice(0), msg_v[0], sems[0])

        def phase(k, s):
            @pl.when(k < cnt)
            def _():
                @pl.when(k + 1 < cnt)
                def _():
                    pltpu.async_copy(msg_slice(k + 1), msg_v[1 - s],
                                     sems[1 - s])
                pltpu.make_async_copy(msg_slice(k), msg_v[s], sems[s]).wait()
                pltpu.sync_copy(msg_v[s], acc_sp.at[idx_v.at[k]], add=True)

        def pair_body(k2, _):
            phase(2 * k2, 0)
            phase(2 * k2 + 1, 1)
            return 0

        lax.fori_loop(0, (mrows + 1) // 2, pair_body, 0)
        plsc.subcore_barrier()

        nk2 = ((n // wbc) - sid + NS - 1) // NS

        def wb_body(k, _):
            start = (sid + k * NS) * wbc
            pltpu.sync_copy(acc_sp.at[pl.ds(start, wbc)], tmp_v)
            pltpu.sync_copy(tmp_v,
                            aggr_hbm.at[pl.ds(start, wbc), pl.ds(colbase, cw)])
            return 0

        lax.fori_loop(0, nk2, wb_body, 0)

    return sc_scatter


def _node_body(xu_ref, aggr0_ref, aggr1_ref, u1b_ref, u2_ref, ub2_ref,
               out_ref):
    ag = aggr0_ref[...] + aggr1_ref[...]
    u = xu_ref[...] + jnp.dot(ag, u1b_ref[...],
                              preferred_element_type=jnp.float32)
    u = u * jax.nn.sigmoid(u)
    out_ref[...] = (jnp.dot(u, u2_ref[...], preferred_element_type=jnp.float32)
                    + ub2_ref[...])


def _tc_node(xu, aggr0, aggr1, u1b, u2, ub2, blk):
    n, h = xu.shape
    grid = n // blk
    row = pl.BlockSpec((blk, h), lambda i: (i, 0))
    full = pl.BlockSpec((h, h), lambda i: (0, 0))
    vec = pl.BlockSpec((1, h), lambda i: (0, 0))
    return pl.pallas_call(
        _node_body,
        grid=(grid,),
        in_specs=[row, row, row, full, full, vec],
        out_specs=row,
        out_shape=jax.ShapeDtypeStruct((n, h), jnp.float32),
    )(xu, aggr0, aggr1, u1b, u2, ub2.reshape(1, h))


def kernel(x, pos, edge_index, W1, b1, W2, b2, U1, ub1, U2, ub2):
    n, h = x.shape
    e = edge_index.shape[1]
    send = edge_index[0].astype(jnp.int32)
    rec = edge_index[1].astype(jnp.int32)

    w1a = W1[:h]
    w1b = W1[h:2 * h]
    w1c = W1[2 * h]
    u1a = U1[:h]
    u1b = U1[h:]

    a_tab, b_tab, xu = _tc_pre(x, w1a, w1b, b1, u1a, ub1, blk=1000)
    px, py, pz = pos[:, 0], pos[:, 1], pos[:, 2]

    es = e // 2
    gather = _sc_gather_build(n, es, h, c=128)
    scatter = _sc_scatter_build(n, es, h, c=128)
    aggrs = []
    nch = es // 128
    mrows = -(-nch // NS)
    mrows = -(-mrows // 8) * 8 if mrows % 8 else mrows
    pad_rows = mrows * NS - nch
    rec2ds = [
        jnp.pad(rec[slice(s * es, (s + 1) * es)].reshape(nch, 128),
                ((0, pad_rows), (0, 0))) for s in range(2)]
    for s in range(2):
        sl = slice(s * es, (s + 1) * es)
        pre0, d2 = gather(a_tab, b_tab, px, py, pz, send[sl], rec[sl])
        msg = _tc_edge(pre0, d2, w1c, W2, b2, blk=640)
        aggrs.append(scatter(msg, rec2ds[s]))
    return _tc_node(xu, aggrs[0], aggrs[1], u1b, U2, ub2, blk=1000)


# speedup vs baseline: 1.2832x; 1.0492x over previous
import functools

import jax
import jax.numpy as jnp
from jax import lax
from jax.experimental import pallas as pl
from jax.experimental.pallas import tpu as pltpu
from jax.experimental.pallas import tpu_sc as plsc

NC = 2
NS = 16
L = 16
NW = NC * NS


def _pre_body(x_ref, wa_ref, wb_ref, b1_ref, ua_ref, ub1_ref,
              a_ref, b_ref, xu_ref):
    xb = x_ref[...]
    a_ref[...] = jnp.dot(xb, wa_ref[...], preferred_element_type=jnp.float32)
    b_ref[...] = (jnp.dot(xb, wb_ref[...], preferred_element_type=jnp.float32)
                  + b1_ref[...])
    xu_ref[...] = (jnp.dot(xb, ua_ref[...], preferred_element_type=jnp.float32)
                   + ub1_ref[...])


def _tc_pre(x, w1a, w1b, b1, u1a, ub1, blk):
    n, h = x.shape
    grid = n // blk
    row = pl.BlockSpec((blk, h), lambda i: (i, 0))
    full = pl.BlockSpec((h, h), lambda i: (0, 0))
    vec = pl.BlockSpec((1, h), lambda i: (0, 0))
    out = jax.ShapeDtypeStruct((n, h), jnp.float32)
    return pl.pallas_call(
        _pre_body,
        grid=(grid,),
        in_specs=[row, full, full, vec, full, vec],
        out_specs=[row, row, row],
        out_shape=[out, out, out],
    )(x, w1a, w1b, b1.reshape(1, h), u1a, ub1.reshape(1, h))


def _sc_gather_build(n, e, h, c):
    nchunks = e // c
    mesh = plsc.VectorSubcoreMesh(core_axis_name="c", subcore_axis_name="s")

    @functools.partial(
        pl.kernel,
        mesh=mesh,
        out_type=(jax.ShapeDtypeStruct((e, h), jnp.float32),
                  jax.ShapeDtypeStruct((e,), jnp.float32)),
        scratch_types=[
            pltpu.VMEM((c,), jnp.int32),
            pltpu.VMEM((c,), jnp.int32),
            pltpu.VMEM((c, h), jnp.float32),
            pltpu.VMEM((c, h), jnp.float32),
            pltpu.VMEM((c,), jnp.float32),
            [pltpu.VMEM((c,), jnp.float32) for _ in range(6)],
            pltpu.SemaphoreType.DMA,
            pltpu.SemaphoreType.DMA,
            pltpu.SemaphoreType.DMA,
        ],
    )
    def sc_gather(a_hbm, b_hbm, px_hbm, py_hbm, pz_hbm, send_hbm, rec_hbm,
                  pre0_hbm, d2_hbm,
                  sidx_v, ridx_v, bufa, bufb, d2_v, coord_v,
                  sem_a, sem_b, sem_p):
        wid = lax.axis_index("s") * NC + lax.axis_index("c")
        nk = (nchunks - wid + NW - 1) // NW

        def chunk_body(k, _):
            cid = wid + k * NW
            base = cid * c
            pltpu.sync_copy(send_hbm.at[pl.ds(base, c)], sidx_v)
            pltpu.sync_copy(rec_hbm.at[pl.ds(base, c)], ridx_v)
            cp_a = pltpu.async_copy(a_hbm.at[sidx_v], bufa, sem_a)
            cp_b = pltpu.async_copy(b_hbm.at[ridx_v], bufb, sem_b)
            cps = []
            for t, (tab, idx) in enumerate(
                    ((px_hbm, sidx_v), (py_hbm, sidx_v), (pz_hbm, sidx_v),
                     (px_hbm, ridx_v), (py_hbm, ridx_v), (pz_hbm, ridx_v))):
                cps.append(pltpu.async_copy(tab.at[idx], coord_v[t], sem_p))
            cp_a.wait()
            cp_b.wait()

            def row_body(i, _):
                for j in range(h // L):
                    sl = pl.ds(j * L, L)
                    bufa[i, sl] = bufa[i, sl] + bufb[i, sl]
                return 0

            lax.fori_loop(0, c, row_body, 0)

            for cp in cps:
                cp.wait()
            for g in range(c // L):
                sl = pl.ds(g * L, L)
                dx = coord_v[0][sl] - coord_v[3][sl]
                dy = coord_v[1][sl] - coord_v[4][sl]
                dz = coord_v[2][sl] - coord_v[5][sl]
                d2_v[sl] = dx * dx + dy * dy + dz * dz

            pltpu.sync_copy(bufa, pre0_hbm.at[pl.ds(base, c)])
            pltpu.sync_copy(d2_v, d2_hbm.at[pl.ds(base, c)])
            return 0

        lax.fori_loop(0, nk, chunk_body, 0)

    return sc_gather


def _edge_body(pre0_ref, d2_ref, w1c_ref, w2_ref, b2_ref, msg_ref):
    pre1 = (pre0_ref[...].astype(jnp.float32)
            + jnp.sqrt(d2_ref[...]) * w1c_ref[...])
    hmid = pre1 * jax.nn.sigmoid(pre1)
    m = (jnp.dot(hmid.astype(jnp.bfloat16), w2_ref[...],
                 preferred_element_type=jnp.float32) + b2_ref[...])
    msg_ref[...] = m * jax.nn.sigmoid(m)


def _tc_edge(pre0, d2, w1c, w2, b2, blk):
    e, h = pre0.shape
    grid = e // blk
    row = pl.BlockSpec((blk, h), lambda i: (i, 0))
    col = pl.BlockSpec((blk, 1), lambda i: (i, 0))
    full = pl.BlockSpec((h, h), lambda i: (0, 0))
    vec = pl.BlockSpec((1, h), lambda i: (0, 0))
    return pl.pallas_call(
        _edge_body,
        grid=(grid,),
        in_specs=[row, col, vec, full, vec],
        out_specs=row,
        out_shape=jax.ShapeDtypeStruct((e, h), jnp.float32),
    )(pre0, d2.reshape(e, 1), w1c.reshape(1, h),
      w2.astype(jnp.bfloat16), b2.reshape(1, h))


def _sc_scatter_build(n, e, h, c):
    nchunks = e // c
    mrows = -(-nchunks // NS)
    mrows = -(-mrows // 8) * 8 if mrows % 8 else mrows
    cw = h // NC
    rpt = n // NS
    wb = 25
    wbc = 40
    mesh = plsc.VectorSubcoreMesh(core_axis_name="c", subcore_axis_name="s")

    @functools.partial(
        pl.kernel,
        mesh=mesh,
        out_type=jax.ShapeDtypeStruct((n, h), jnp.float32),
        scratch_types=[
            pltpu.VMEM((mrows, c), jnp.int32),
            [pltpu.VMEM((c, cw), jnp.float32) for _ in range(2)],
            pltpu.VMEM((wbc, cw), jnp.float32),
            pltpu.VMEM_SHARED((n, cw), jnp.float32),
            pltpu.SemaphoreType.DMA,
            pltpu.SemaphoreType.DMA,
        ],
    )
    def sc_scatter(msg_hbm, rec2d_hbm, aggr_hbm,
                   idx_v, msg_v, tmp_v, acc_sp, sem0, sem1):
        core = lax.axis_index("c")
        sid = lax.axis_index("s")
        colbase = pl.multiple_of(core * cw, cw)
        start = pl.multiple_of(sid * mrows, 8)
        cnt = jnp.clip(nchunks - sid * mrows, 0, mrows)
        sems = (sem0, sem1)

        pltpu.sync_copy(rec2d_hbm.at[pl.ds(start, mrows)], idx_v)

        def zrow(i, _):
            for j in range(cw // L):
                tmp_v[i, pl.ds(j * L, L)] = jnp.zeros((L,), jnp.float32)
            return 0

        lax.fori_loop(0, wb, zrow, 0)

        def zcopy(t, _):
            pltpu.sync_copy(tmp_v.at[pl.ds(0, wb)],
                            acc_sp.at[pl.ds(sid * rpt + t * wb, wb)])
            return 0

        lax.fori_loop(0, rpt // wb, zcopy, 0)
        plsc.subcore_barrier()

        def msg_slice(k):
            return msg_hbm.at[pl.ds((start + k) * c, c), pl.ds(colbase, cw)]

        @pl.when(cnt > 0)
        def _():
            pltpu.async_copy(msg_slice(0), msg_v[0], sems[0])

        def phase(k, s):
            @pl.when(k < cnt)
            def _():
                @pl.when(k + 1 < cnt)
                def _():
                    pltpu.async_copy(msg_slice(k + 1), msg_v[1 - s],
                                     sems[1 - s])
                pltpu.make_async_copy(msg_slice(k), msg_v[s], sems[s]).wait()
                pltpu.sync_copy(msg_v[s], acc_sp.at[idx_v.at[k]], add=True)

        def pair_body(k2, _):
            phase(2 * k2, 0)
            phase(2 * k2 + 1, 1)
            return 0

        lax.fori_loop(0, (mrows + 1) // 2, pair_body, 0)
        plsc.subcore_barrier()

        nk2 = ((n // wbc) - sid + NS - 1) // NS

        def wb_body(k, _):
            start = (sid + k * NS) * wbc
            pltpu.sync_copy(acc_sp.at[pl.ds(start, wbc)], tmp_v)
            pltpu.sync_copy(tmp_v,
                            aggr_hbm.at[pl.ds(start, wbc), pl.ds(colbase, cw)])
            return 0

        lax.fori_loop(0, nk2, wb_body, 0)

    return sc_scatter


def _node_body(*refs):
    xu_ref, *aggr_refs, u1b_ref, u2_ref, ub2_ref, out_ref = refs
    ag = aggr_refs[0][...]
    for r in aggr_refs[1:]:
        ag = ag + r[...]
    u = xu_ref[...] + jnp.dot(ag, u1b_ref[...],
                              preferred_element_type=jnp.float32)
    u = u * jax.nn.sigmoid(u)
    out_ref[...] = (jnp.dot(u, u2_ref[...], preferred_element_type=jnp.float32)
                    + ub2_ref[...])


def _tc_node(xu, aggrs, u1b, u2, ub2, blk):
    n, h = xu.shape
    grid = n // blk
    row = pl.BlockSpec((blk, h), lambda i: (i, 0))
    full = pl.BlockSpec((h, h), lambda i: (0, 0))
    vec = pl.BlockSpec((1, h), lambda i: (0, 0))
    return pl.pallas_call(
        _node_body,
        grid=(grid,),
        in_specs=[row] + [row] * len(aggrs) + [full, full, vec],
        out_specs=row,
        out_shape=jax.ShapeDtypeStruct((n, h), jnp.float32),
    )(xu, *aggrs, u1b, u2, ub2.reshape(1, h))


def kernel(x, pos, edge_index, W1, b1, W2, b2, U1, ub1, U2, ub2):
    n, h = x.shape
    e = edge_index.shape[1]
    send = edge_index[0].astype(jnp.int32)
    rec = edge_index[1].astype(jnp.int32)

    w1a = W1[:h]
    w1b = W1[h:2 * h]
    w1c = W1[2 * h]
    u1a = U1[:h]
    u1b = U1[h:]

    a_tab, b_tab, xu = _tc_pre(x, w1a, w1b, b1, u1a, ub1, blk=1000)
    px, py, pz = pos[:, 0], pos[:, 1], pos[:, 2]

    bounds = [0, 53760, 106880, 160000]
    builds = {}
    aggrs = []
    for s in range(len(bounds) - 1):
        lo, hi = bounds[s], bounds[s + 1]
        es = hi - lo
        if es not in builds:
            nch = es // 128
            mrows = -(-nch // NS)
            mrows = -(-mrows // 8) * 8 if mrows % 8 else mrows
            builds[es] = (_sc_gather_build(n, es, h, c=128),
                          _sc_scatter_build(n, es, h, c=128),
                          mrows * NS - nch, nch)
        gather, scatter, pad_rows, nch = builds[es]
        sl = slice(lo, hi)
        rec2d = jnp.pad(rec[sl].reshape(nch, 128), ((0, pad_rows), (0, 0)))
        pre0, d2 = gather(a_tab, b_tab, px, py, pz, send[sl], rec[sl])
        msg = _tc_edge(pre0, d2, w1c, W2, b2, blk=640)
        aggrs.append(scatter(msg, rec2d))
    return _tc_node(xu, aggrs, u1b, U2, ub2, blk=1000)
